# baseline (speedup 1.0000x reference)
import functools

import jax
import jax.numpy as jnp
from jax.experimental import pallas as pl
from jax.experimental.pallas import tpu as pltpu

N = 4096
TILE = 512
GRID = N // TILE


def _pairwise_body(tc_ref, tr_ref, pc_ref, pr_ref, loss_ref, rev_ref, acc_ref):
    i = pl.program_id(0)
    ti = tc_ref[...]
    tj = tr_ref[...]
    pi = pc_ref[...]
    pj = pr_ref[...]

    diff = (ti - tj) / (jnp.abs(tj) + 0.0001)
    valid = diff > 2.0
    vf = jnp.where(valid, 1.0, 0.0)
    cnt = jnp.sum(vf)
    rev = jnp.sum(jnp.where(valid & ((pi - pj) > 0.0), 1.0, 0.0))
    term = jnp.log(1.0 + jnp.exp(0.5 * (pj - pi + 1.0)))
    ls = jnp.sum(jnp.where(valid, term, 0.0))

    @pl.when(i == 0)
    def _init():
        acc_ref[0] = ls
        acc_ref[1] = cnt
        acc_ref[2] = rev

    @pl.when(i > 0)
    def _accum():
        acc_ref[0] += ls
        acc_ref[1] += cnt
        acc_ref[2] += rev

    @pl.when(i == GRID - 1)
    def _finalize():
        num = acc_ref[1] + 1e-8
        loss_ref[...] = jnp.full((1, 1), acc_ref[0] / num, dtype=jnp.float32)
        rev_ref[...] = jnp.full((1, 1), acc_ref[2] / num, dtype=jnp.float32)


@jax.jit
def kernel(pred, true):
    tc = true.reshape(N, 1)
    tr = true.reshape(1, N)
    pc = pred.reshape(N, 1)
    pr = pred.reshape(1, N)
    loss, rev = pl.pallas_call(
        _pairwise_body,
        grid=(GRID,),
        in_specs=[
            pl.BlockSpec((TILE, 1), lambda i: (i, 0)),
            pl.BlockSpec((1, N), lambda i: (0, 0)),
            pl.BlockSpec((TILE, 1), lambda i: (i, 0)),
            pl.BlockSpec((1, N), lambda i: (0, 0)),
        ],
        out_specs=[
            pl.BlockSpec((1, 1), lambda i: (0, 0)),
            pl.BlockSpec((1, 1), lambda i: (0, 0)),
        ],
        out_shape=[
            jax.ShapeDtypeStruct((1, 1), jnp.float32),
            jax.ShapeDtypeStruct((1, 1), jnp.float32),
        ],
        scratch_shapes=[pltpu.SMEM((3,), jnp.float32)],
    )(tc, tr, pc, pr)
    return (loss.reshape(()), rev.reshape(()))


# speedup vs baseline: 1.1966x; 1.1966x over previous
import functools

import jax
import jax.numpy as jnp
from jax.experimental import pallas as pl
from jax.experimental.pallas import tpu as pltpu

N = 4096
TILE = 512
GRID = N // TILE


_LOG2E = 1.4426950408889634
_LN2 = 0.6931471805599453
_C = 0.5 * _LOG2E


def _pairwise_body(tc_ref, tr_ref, pc_ref, pr_ref, loss_ref, rev_ref, acc_ref):
    i = pl.program_id(0)
    ti = tc_ref[...]
    tj = tr_ref[...]
    pi = pc_ref[...]
    pj = pr_ref[...]

    thr_j = tj + 2.0 * (jnp.abs(tj) + 0.0001)
    a2_j = _C * pj + _C
    b2_i = _C * pi

    valid = ti > thr_j
    vf = jnp.where(valid, 1.0, 0.0)
    cnt = jnp.sum(vf)
    rev = jnp.sum(jnp.where(valid & (pi > pj), 1.0, 0.0))
    term2 = jnp.log2(1.0 + jnp.exp2(a2_j - b2_i))
    ls = _LN2 * jnp.sum(jnp.where(valid, term2, 0.0))

    @pl.when(i == 0)
    def _init():
        acc_ref[0] = ls
        acc_ref[1] = cnt
        acc_ref[2] = rev

    @pl.when(i > 0)
    def _accum():
        acc_ref[0] += ls
        acc_ref[1] += cnt
        acc_ref[2] += rev

    @pl.when(i == GRID - 1)
    def _finalize():
        num = acc_ref[1] + 1e-8
        loss_ref[...] = jnp.full((1, 1), acc_ref[0] / num, dtype=jnp.float32)
        rev_ref[...] = jnp.full((1, 1), acc_ref[2] / num, dtype=jnp.float32)


@jax.jit
def kernel(pred, true):
    tc = true.reshape(N, 1)
    tr = true.reshape(1, N)
    pc = pred.reshape(N, 1)
    pr = pred.reshape(1, N)
    loss, rev = pl.pallas_call(
        _pairwise_body,
        grid=(GRID,),
        in_specs=[
            pl.BlockSpec((TILE, 1), lambda i: (i, 0)),
            pl.BlockSpec((1, N), lambda i: (0, 0)),
            pl.BlockSpec((TILE, 1), lambda i: (i, 0)),
            pl.BlockSpec((1, N), lambda i: (0, 0)),
        ],
        out_specs=[
            pl.BlockSpec((1, 1), lambda i: (0, 0)),
            pl.BlockSpec((1, 1), lambda i: (0, 0)),
        ],
        out_shape=[
            jax.ShapeDtypeStruct((1, 1), jnp.float32),
            jax.ShapeDtypeStruct((1, 1), jnp.float32),
        ],
        scratch_shapes=[pltpu.SMEM((3,), jnp.float32)],
    )(tc, tr, pc, pr)
    return (loss.reshape(()), rev.reshape(()))


# speedup vs baseline: 1.3550x; 1.1323x over previous
import functools

import jax
import jax.numpy as jnp
from jax.experimental import pallas as pl
from jax.experimental.pallas import tpu as pltpu

N = 4096
TILE = 512
GRID = N // TILE


_LOG2E = 1.4426950408889634
_LN2 = 0.6931471805599453
_C = 0.5 * _LOG2E


def _pairwise_body(tc_ref, tr_ref, pc_ref, pr_ref, loss_ref, rev_ref, acc_ref):
    i = pl.program_id(0)
    ti = tc_ref[...]
    tj = tr_ref[...]
    pi = pc_ref[...]
    pj = pr_ref[...]

    thr_j = tj + 2.0 * (jnp.abs(tj) + 0.0001)
    e_j = jnp.exp2(_C * pj + _C)
    f_i = jnp.exp2(-_C * pi)

    vf = jnp.where(ti > thr_j, 1.0, 0.0)
    rev_f = jnp.where(pi > pj, vf, 0.0)
    cnt = jnp.sum(vf)
    rev = jnp.sum(rev_f)
    term2 = jnp.log2(1.0 + e_j * f_i)
    ls = _LN2 * jnp.sum(term2 * vf)

    @pl.when(i == 0)
    def _init():
        acc_ref[0] = ls
        acc_ref[1] = cnt
        acc_ref[2] = rev

    @pl.when(i > 0)
    def _accum():
        acc_ref[0] += ls
        acc_ref[1] += cnt
        acc_ref[2] += rev

    @pl.when(i == GRID - 1)
    def _finalize():
        num = acc_ref[1] + 1e-8
        loss_ref[...] = jnp.full((1, 1), acc_ref[0] / num, dtype=jnp.float32)
        rev_ref[...] = jnp.full((1, 1), acc_ref[2] / num, dtype=jnp.float32)


@jax.jit
def kernel(pred, true):
    tc = true.reshape(N, 1)
    tr = true.reshape(1, N)
    pc = pred.reshape(N, 1)
    pr = pred.reshape(1, N)
    loss, rev = pl.pallas_call(
        _pairwise_body,
        grid=(GRID,),
        in_specs=[
            pl.BlockSpec((TILE, 1), lambda i: (i, 0)),
            pl.BlockSpec((1, N), lambda i: (0, 0)),
            pl.BlockSpec((TILE, 1), lambda i: (i, 0)),
            pl.BlockSpec((1, N), lambda i: (0, 0)),
        ],
        out_specs=[
            pl.BlockSpec((1, 1), lambda i: (0, 0)),
            pl.BlockSpec((1, 1), lambda i: (0, 0)),
        ],
        out_shape=[
            jax.ShapeDtypeStruct((1, 1), jnp.float32),
            jax.ShapeDtypeStruct((1, 1), jnp.float32),
        ],
        scratch_shapes=[pltpu.SMEM((3,), jnp.float32)],
    )(tc, tr, pc, pr)
    return (loss.reshape(()), rev.reshape(()))


# speedup vs baseline: 1.7431x; 1.2865x over previous
import functools

import jax
import jax.numpy as jnp
from jax.experimental import pallas as pl
from jax.experimental.pallas import tpu as pltpu

N = 4096
TILE = 512
GRID = N // TILE


_LOG2E = 1.4426950408889634
_LN2 = 0.6931471805599453
_C = 0.5 * _LOG2E


def _pairwise_body(tc_ref, tr_ref, pc_ref, pr_ref, loss_ref, rev_ref, acc_ref):
    i = pl.program_id(0)
    ti = tc_ref[...]
    tj = tr_ref[...]
    pi = pc_ref[...]
    pj = pr_ref[...]

    thr_j = tj + 2.0 * (jnp.abs(tj) + 0.0001)
    e_j = jnp.exp2(_C * pj + _C)
    f_i = jnp.exp2(-_C * pi)

    vf = jnp.where(ti > thr_j, 1.0, 0.0)
    rev_f = jnp.where(pi > pj, vf, 0.0)
    lmat = jnp.log2(1.0 + e_j * f_i) * vf
    ones_col = jnp.ones((N, 1), dtype=jnp.float32)
    cnt = jnp.sum(jnp.dot(vf, ones_col, preferred_element_type=jnp.float32))
    rev = jnp.sum(jnp.dot(rev_f, ones_col, preferred_element_type=jnp.float32))
    ls = _LN2 * jnp.sum(jnp.dot(lmat, ones_col, preferred_element_type=jnp.float32))

    @pl.when(i == 0)
    def _init():
        acc_ref[0] = ls
        acc_ref[1] = cnt
        acc_ref[2] = rev

    @pl.when(i > 0)
    def _accum():
        acc_ref[0] += ls
        acc_ref[1] += cnt
        acc_ref[2] += rev

    @pl.when(i == GRID - 1)
    def _finalize():
        num = acc_ref[1] + 1e-8
        loss_ref[...] = jnp.full((1, 1), acc_ref[0] / num, dtype=jnp.float32)
        rev_ref[...] = jnp.full((1, 1), acc_ref[2] / num, dtype=jnp.float32)


@jax.jit
def kernel(pred, true):
    tc = true.reshape(N, 1)
    tr = true.reshape(1, N)
    pc = pred.reshape(N, 1)
    pr = pred.reshape(1, N)
    loss, rev = pl.pallas_call(
        _pairwise_body,
        grid=(GRID,),
        in_specs=[
            pl.BlockSpec((TILE, 1), lambda i: (i, 0)),
            pl.BlockSpec((1, N), lambda i: (0, 0)),
            pl.BlockSpec((TILE, 1), lambda i: (i, 0)),
            pl.BlockSpec((1, N), lambda i: (0, 0)),
        ],
        out_specs=[
            pl.BlockSpec((1, 1), lambda i: (0, 0)),
            pl.BlockSpec((1, 1), lambda i: (0, 0)),
        ],
        out_shape=[
            jax.ShapeDtypeStruct((1, 1), jnp.float32),
            jax.ShapeDtypeStruct((1, 1), jnp.float32),
        ],
        scratch_shapes=[pltpu.SMEM((3,), jnp.float32)],
    )(tc, tr, pc, pr)
    return (loss.reshape(()), rev.reshape(()))
